# baseline (speedup 1.0000x reference)
import functools

import jax
import jax.numpy as jnp
from jax import lax
from jax.experimental import pallas as pl
from jax.experimental.pallas import tpu as pltpu
from jax.experimental.pallas import tpu_sc as plsc

N_NODES = 10000
N_EDGES = 320000
D_IN = 128
F_SIZE = 64
OUT_SIZE = 16
NUM_GRAPHS = 64

NC = 2
NS = 16
NW = NC * NS
CHUNK = 128
CPW = 80
EPO = 256
NBUF = 2
TOT_CH = 1280
NCH0 = 80
NCH1 = 0
NCHMAX = max(NCH0, NCH1)
E_PAD = NW * CPW * CHUNK
NROWS = 10112
ROWS_PER_TILE = NROWS // NS
DEG_W = 16

_MESH = plsc.VectorSubcoreMesh(core_axis_name="c", subcore_axis_name="s")


@functools.partial(
    pl.kernel,
    out_type=jax.ShapeDtypeStruct((NC, NROWS, DEG_W), jnp.float32),
    mesh=_MESH,
    scratch_types=[
        pltpu.VMEM((CPW, CHUNK), jnp.int32),
        pltpu.VMEM((CHUNK, DEG_W), jnp.float32),
        pltpu.VMEM_SHARED((NROWS, DEG_W), jnp.float32),
    ],
    compiler_params=pltpu.CompilerParams(use_tc_tiling_on_sc=False),
)
def _sc_degree(dst_hbm, ones_hbm, zeros_hbm, out_hbm, dst_v, ones_v, acc_sh):
    cid = lax.axis_index("c")
    sid = lax.axis_index("s")
    wid = cid * NS + sid
    pltpu.sync_copy(
        zeros_hbm.at[pl.ds(pl.multiple_of(sid * ROWS_PER_TILE, 8), ROWS_PER_TILE)],
        acc_sh.at[pl.ds(pl.multiple_of(sid * ROWS_PER_TILE, 8), ROWS_PER_TILE)],
    )
    pltpu.sync_copy(ones_hbm, ones_v)
    pltpu.sync_copy(dst_hbm.at[wid], dst_v)
    plsc.subcore_barrier()

    def body(j, carry):
        pltpu.sync_copy(ones_v, acc_sh.at[dst_v.at[j]], add=True)
        return carry

    lax.fori_loop(0, CPW, body, 0)
    plsc.subcore_barrier()
    pltpu.sync_copy(
        acc_sh.at[pl.ds(pl.multiple_of(sid * ROWS_PER_TILE, 8), ROWS_PER_TILE)],
        out_hbm.at[cid, pl.ds(pl.multiple_of(sid * ROWS_PER_TILE, 8), ROWS_PER_TILE)],
    )


@functools.partial(
    pl.kernel,
    out_type=jax.ShapeDtypeStruct((NC, NROWS, F_SIZE), jnp.float32),
    mesh=_MESH,
    scratch_types=[
        pltpu.VMEM((NCHMAX, EPO), jnp.int32),
        pltpu.VMEM((NCHMAX, EPO), jnp.int32),
        [pltpu.VMEM((EPO, F_SIZE), jnp.float32) for _ in range(NBUF)],
        pltpu.VMEM_SHARED((NROWS, F_SIZE), jnp.float32),
        [pltpu.SemaphoreType.DMA for _ in range(NBUF)],
        [pltpu.SemaphoreType.DMA for _ in range(NBUF)],
    ],
    compiler_params=pltpu.CompilerParams(use_tc_tiling_on_sc=False),
)
def _sc_agg(h_hbm, src_hbm, dst_hbm, zeros_hbm, out_hbm,
            src_v, dst_v, rows, acc_sh, gsem, ssem):
    cid = lax.axis_index("c")
    sid = lax.axis_index("s")
    pltpu.sync_copy(
        zeros_hbm.at[pl.ds(pl.multiple_of(sid * ROWS_PER_TILE, 8), ROWS_PER_TILE)],
        acc_sh.at[pl.ds(pl.multiple_of(sid * ROWS_PER_TILE, 8), ROWS_PER_TILE)],
    )

    def run(base, nch):
        pltpu.sync_copy(src_hbm.at[pl.ds(base, nch)], src_v.at[pl.ds(0, nch)])
        pltpu.sync_copy(dst_hbm.at[pl.ds(base, nch)], dst_v.at[pl.ds(0, nch)])
        for b in range(NBUF):
            pltpu.async_copy(h_hbm.at[src_v.at[b]], rows[b], gsem[b])

        def body(i, carry):
            for b in range(NBUF):
                j = NBUF * i + b
                pltpu.make_async_copy(
                    h_hbm.at[src_v.at[j]], rows[b], gsem[b]).wait()
                pltpu.async_copy(rows[b], acc_sh.at[dst_v.at[j]], ssem[b],
                                 add=True)
            for b in range(NBUF):
                j = NBUF * i + b

                @pl.when(j + NBUF < nch)
                def _(b=b, j=j):
                    pltpu.make_async_copy(
                        rows[b], acc_sh.at[dst_v.at[j]], ssem[b]).wait()
                    pltpu.async_copy(h_hbm.at[src_v.at[j + NBUF]], rows[b],
                                     gsem[b])

            return carry

        lax.fori_loop(0, nch // NBUF, body, 0)
        for b in range(NBUF):
            j = nch - NBUF + b
            pltpu.make_async_copy(rows[b], acc_sh.at[dst_v.at[j]],
                                  ssem[b]).wait()

    if NCH0:
        @pl.when(cid == 0)
        def _():
            run(pl.multiple_of(sid * NCH0, 8), NCH0)

    if NCH1:
        @pl.when(cid == 1)
        def _():
            run(pl.multiple_of(NS * NCH0 + sid * NCH1, 8), NCH1)

    plsc.subcore_barrier()
    pltpu.sync_copy(
        acc_sh.at[pl.ds(pl.multiple_of(sid * ROWS_PER_TILE, 8), ROWS_PER_TILE)],
        out_hbm.at[cid, pl.ds(pl.multiple_of(sid * ROWS_PER_TILE, 8), ROWS_PER_TILE)],
    )


def _tc_prologue(deg_ref, x_ref, w1_ref, dinv_ref, h1_ref):
    deg = deg_ref[0, :, 0:1] + deg_ref[1, :, 0:1] + 1.0
    rows = lax.broadcasted_iota(jnp.int32, (NROWS, 1), 0)
    dinv = jnp.where(rows < N_NODES, lax.rsqrt(deg), 0.0)
    dinv_ref[...] = dinv
    h1_ref[...] = jnp.dot(x_ref[...], w1_ref[...],
                          preferred_element_type=jnp.float32) * dinv


def _tc_layer(agg_ref, h_ref, dinv_ref, b_ref, wn_ref, x_ref, hn_ref):
    dinv = dinv_ref[...]
    out = (agg_ref[0] + agg_ref[1] + h_ref[...]) * dinv + b_ref[...]
    x = jnp.maximum(out, 0.0)
    x_ref[...] = x
    hn_ref[...] = jnp.dot(x, wn_ref[...],
                          preferred_element_type=jnp.float32) * dinv


def _tc_final(agg_ref, h3_ref, dinv_ref, b3_ref, x1_ref, x2_ref,
              batch_ref, wf_ref, bf_ref, out_ref):
    dinv = dinv_ref[...]
    x3 = jnp.maximum(
        (agg_ref[0] + agg_ref[1] + h3_ref[...]) * dinv + b3_ref[...], 0.0)
    onehot = (batch_ref[...] ==
              lax.broadcasted_iota(jnp.int32, (NUM_GRAPHS, N_NODES), 0))
    seg = jnp.where(onehot, 1.0, 0.0)
    counts = jnp.sum(seg, axis=1, keepdims=True)
    hcat = jnp.concatenate(
        [x1_ref[:N_NODES], x2_ref[:N_NODES], x3[:N_NODES]], axis=1)
    pooled = jnp.dot(seg, hcat, preferred_element_type=jnp.float32)
    pooled = pooled / jnp.maximum(counts, 1.0)
    logits = jnp.dot(pooled, wf_ref[...],
                     preferred_element_type=jnp.float32) + bf_ref[...]
    m = jnp.max(logits, axis=1, keepdims=True)
    e = jnp.exp(logits - m)
    out_ref[...] = e / jnp.sum(e, axis=1, keepdims=True)


def _tc_call(body, out_shape, *args):
    return pl.pallas_call(body, out_shape=out_shape)(*args)


def kernel(x, edge_index, batch, W1, b1, W2, b2, W3, b3, Wf, bf):
    src = edge_index[0].astype(jnp.int32)
    dst = edge_index[1].astype(jnp.int32)
    pad = jnp.full((E_PAD - N_EDGES,), N_NODES, jnp.int32)
    src_pad = jnp.concatenate([src, pad])
    dst_pad = jnp.concatenate([dst, pad])
    src3d = src_pad.reshape(TOT_CH, EPO)
    dst3d = dst_pad.reshape(TOT_CH, EPO)
    dst2d = dst_pad.reshape(NW, CPW, CHUNK)

    x_pad = jnp.zeros((NROWS, D_IN), jnp.float32).at[:N_NODES].set(x)
    zeros_deg = jnp.zeros((NROWS, DEG_W), jnp.float32)
    zeros_f = jnp.zeros((NROWS, F_SIZE), jnp.float32)
    ones_rows = jnp.ones((CHUNK, DEG_W), jnp.float32)

    deg_parts = _sc_degree(dst2d, ones_rows, zeros_deg)

    f32 = jnp.float32
    dinv, h1 = _tc_call(
        _tc_prologue,
        (jax.ShapeDtypeStruct((NROWS, 1), f32),
         jax.ShapeDtypeStruct((NROWS, F_SIZE), f32)),
        deg_parts, x_pad, W1)

    agg1 = _sc_agg(h1, src3d, dst3d, zeros_f)
    x1, h2 = _tc_call(
        _tc_layer,
        (jax.ShapeDtypeStruct((NROWS, F_SIZE), f32),
         jax.ShapeDtypeStruct((NROWS, F_SIZE), f32)),
        agg1, h1, dinv, b1.reshape(1, F_SIZE), W2)

    agg2 = _sc_agg(h2, src3d, dst3d, zeros_f)
    x2, h3 = _tc_call(
        _tc_layer,
        (jax.ShapeDtypeStruct((NROWS, F_SIZE), f32),
         jax.ShapeDtypeStruct((NROWS, F_SIZE), f32)),
        agg2, h2, dinv, b2.reshape(1, F_SIZE), W3)

    agg3 = _sc_agg(h3, src3d, dst3d, zeros_f)
    out = _tc_call(
        _tc_final,
        jax.ShapeDtypeStruct((NUM_GRAPHS, OUT_SIZE), f32),
        agg3, h3, dinv, b3.reshape(1, F_SIZE), x1, x2,
        batch.astype(jnp.int32).reshape(1, N_NODES), Wf,
        bf.reshape(1, OUT_SIZE))
    return out


# speedup vs baseline: 1.3550x; 1.3550x over previous
import functools

import jax
import jax.numpy as jnp
from jax import lax
from jax.experimental import pallas as pl
from jax.experimental.pallas import tpu as pltpu
from jax.experimental.pallas import tpu_sc as plsc

N_NODES = 10000
N_EDGES = 320000
D_IN = 128
F_SIZE = 64
OUT_SIZE = 16
NUM_GRAPHS = 64

NC = 2
NS = 16
NW = NC * NS
CHUNK = 128
CPW = 80
EPO = 256
NBUF = 2
TOT_CH = 1280
NCH0 = 72
NCH1 = 8
NCHMAX = max(NCH0, NCH1)
E_PAD = NW * CPW * CHUNK
NROWS = 10112
ROWS_PER_TILE = NROWS // NS
DEG_W = 16

_MESH = plsc.VectorSubcoreMesh(core_axis_name="c", subcore_axis_name="s")


@functools.partial(
    pl.kernel,
    out_type=jax.ShapeDtypeStruct((NC, NROWS, DEG_W), jnp.float32),
    mesh=_MESH,
    scratch_types=[
        pltpu.VMEM((CPW, CHUNK), jnp.int32),
        pltpu.VMEM((CHUNK, DEG_W), jnp.float32),
        pltpu.VMEM_SHARED((NROWS, DEG_W), jnp.float32),
    ],
    compiler_params=pltpu.CompilerParams(use_tc_tiling_on_sc=False),
)
def _sc_degree(dst_hbm, ones_hbm, zeros_hbm, out_hbm, dst_v, ones_v, acc_sh):
    cid = lax.axis_index("c")
    sid = lax.axis_index("s")
    wid = cid * NS + sid
    pltpu.sync_copy(
        zeros_hbm.at[pl.ds(pl.multiple_of(sid * ROWS_PER_TILE, 8), ROWS_PER_TILE)],
        acc_sh.at[pl.ds(pl.multiple_of(sid * ROWS_PER_TILE, 8), ROWS_PER_TILE)],
    )
    pltpu.sync_copy(ones_hbm, ones_v)
    pltpu.sync_copy(dst_hbm.at[wid], dst_v)
    plsc.subcore_barrier()

    def body(j, carry):
        pltpu.sync_copy(ones_v, acc_sh.at[dst_v.at[j]], add=True)
        return carry

    lax.fori_loop(0, CPW, body, 0)
    plsc.subcore_barrier()
    pltpu.sync_copy(
        acc_sh.at[pl.ds(pl.multiple_of(sid * ROWS_PER_TILE, 8), ROWS_PER_TILE)],
        out_hbm.at[cid, pl.ds(pl.multiple_of(sid * ROWS_PER_TILE, 8), ROWS_PER_TILE)],
    )


@functools.partial(
    pl.kernel,
    out_type=jax.ShapeDtypeStruct((NC, NROWS, F_SIZE), jnp.float32),
    mesh=_MESH,
    scratch_types=[
        pltpu.VMEM((NCHMAX, EPO), jnp.int32),
        pltpu.VMEM((NCHMAX, EPO), jnp.int32),
        [pltpu.VMEM((EPO, F_SIZE), jnp.float32) for _ in range(NBUF)],
        pltpu.VMEM_SHARED((NROWS, F_SIZE), jnp.float32),
        [pltpu.SemaphoreType.DMA for _ in range(NBUF)],
        [pltpu.SemaphoreType.DMA for _ in range(NBUF)],
    ],
    compiler_params=pltpu.CompilerParams(use_tc_tiling_on_sc=False),
)
def _sc_agg(h_hbm, src_hbm, dst_hbm, zeros_hbm, out_hbm,
            src_v, dst_v, rows, acc_sh, gsem, ssem):
    cid = lax.axis_index("c")
    sid = lax.axis_index("s")
    pltpu.sync_copy(
        zeros_hbm.at[pl.ds(pl.multiple_of(sid * ROWS_PER_TILE, 8), ROWS_PER_TILE)],
        acc_sh.at[pl.ds(pl.multiple_of(sid * ROWS_PER_TILE, 8), ROWS_PER_TILE)],
    )

    def run(base, nch):
        pltpu.sync_copy(src_hbm.at[pl.ds(base, nch)], src_v.at[pl.ds(0, nch)])
        pltpu.sync_copy(dst_hbm.at[pl.ds(base, nch)], dst_v.at[pl.ds(0, nch)])
        for b in range(NBUF):
            pltpu.async_copy(h_hbm.at[src_v.at[b]], rows[b], gsem[b])

        def body(i, carry):
            for b in range(NBUF):
                j = NBUF * i + b
                pltpu.make_async_copy(
                    h_hbm.at[src_v.at[j]], rows[b], gsem[b]).wait()
                pltpu.async_copy(rows[b], acc_sh.at[dst_v.at[j]], ssem[b],
                                 add=True)
            for b in range(NBUF):
                j = NBUF * i + b

                @pl.when(j + NBUF < nch)
                def _(b=b, j=j):
                    pltpu.make_async_copy(
                        rows[b], acc_sh.at[dst_v.at[j]], ssem[b]).wait()
                    pltpu.async_copy(h_hbm.at[src_v.at[j + NBUF]], rows[b],
                                     gsem[b])

            return carry

        lax.fori_loop(0, nch // NBUF, body, 0)
        for b in range(NBUF):
            j = nch - NBUF + b
            pltpu.make_async_copy(rows[b], acc_sh.at[dst_v.at[j]],
                                  ssem[b]).wait()

    if NCH0:
        @pl.when(cid == 0)
        def _():
            run(pl.multiple_of(sid * NCH0, 8), NCH0)

    if NCH1:
        @pl.when(cid == 1)
        def _():
            run(pl.multiple_of(NS * NCH0 + sid * NCH1, 8), NCH1)

    plsc.subcore_barrier()
    pltpu.sync_copy(
        acc_sh.at[pl.ds(pl.multiple_of(sid * ROWS_PER_TILE, 8), ROWS_PER_TILE)],
        out_hbm.at[cid, pl.ds(pl.multiple_of(sid * ROWS_PER_TILE, 8), ROWS_PER_TILE)],
    )


def _tc_prologue(deg_ref, x_ref, w1_ref, dinv_ref, h1_ref):
    deg = deg_ref[0, :, 0:1] + deg_ref[1, :, 0:1] + 1.0
    rows = lax.broadcasted_iota(jnp.int32, (NROWS, 1), 0)
    dinv = jnp.where(rows < N_NODES, lax.rsqrt(deg), 0.0)
    dinv_ref[...] = dinv
    h1_ref[...] = jnp.dot(x_ref[...], w1_ref[...],
                          preferred_element_type=jnp.float32) * dinv


def _tc_layer(agg_ref, h_ref, dinv_ref, b_ref, wn_ref, x_ref, hn_ref):
    dinv = dinv_ref[...]
    out = (agg_ref[0] + agg_ref[1] + h_ref[...]) * dinv + b_ref[...]
    x = jnp.maximum(out, 0.0)
    x_ref[...] = x
    hn_ref[...] = jnp.dot(x, wn_ref[...],
                          preferred_element_type=jnp.float32) * dinv


def _tc_final(agg_ref, h3_ref, dinv_ref, b3_ref, x1_ref, x2_ref,
              batch_ref, wf_ref, bf_ref, out_ref):
    dinv = dinv_ref[...]
    x3 = jnp.maximum(
        (agg_ref[0] + agg_ref[1] + h3_ref[...]) * dinv + b3_ref[...], 0.0)
    onehot = (batch_ref[...] ==
              lax.broadcasted_iota(jnp.int32, (NUM_GRAPHS, N_NODES), 0))
    seg = jnp.where(onehot, 1.0, 0.0)
    counts = jnp.sum(seg, axis=1, keepdims=True)
    hcat = jnp.concatenate(
        [x1_ref[:N_NODES], x2_ref[:N_NODES], x3[:N_NODES]], axis=1)
    pooled = jnp.dot(seg, hcat, preferred_element_type=jnp.float32)
    pooled = pooled / jnp.maximum(counts, 1.0)
    logits = jnp.dot(pooled, wf_ref[...],
                     preferred_element_type=jnp.float32) + bf_ref[...]
    m = jnp.max(logits, axis=1, keepdims=True)
    e = jnp.exp(logits - m)
    out_ref[...] = e / jnp.sum(e, axis=1, keepdims=True)


def _tc_call(body, out_shape, *args):
    return pl.pallas_call(body, out_shape=out_shape)(*args)


def kernel(x, edge_index, batch, W1, b1, W2, b2, W3, b3, Wf, bf):
    src = edge_index[0].astype(jnp.int32)
    dst = edge_index[1].astype(jnp.int32)
    pad = jnp.full((E_PAD - N_EDGES,), N_NODES, jnp.int32)
    src_pad = jnp.concatenate([src, pad])
    dst_pad = jnp.concatenate([dst, pad])
    src3d = src_pad.reshape(TOT_CH, EPO)
    dst3d = dst_pad.reshape(TOT_CH, EPO)
    dst2d = dst_pad.reshape(NW, CPW, CHUNK)

    x_pad = jnp.zeros((NROWS, D_IN), jnp.float32).at[:N_NODES].set(x)
    zeros_deg = jnp.zeros((NROWS, DEG_W), jnp.float32)
    zeros_f = jnp.zeros((NROWS, F_SIZE), jnp.float32)
    ones_rows = jnp.ones((CHUNK, DEG_W), jnp.float32)

    deg_parts = _sc_degree(dst2d, ones_rows, zeros_deg)

    f32 = jnp.float32
    dinv, h1 = _tc_call(
        _tc_prologue,
        (jax.ShapeDtypeStruct((NROWS, 1), f32),
         jax.ShapeDtypeStruct((NROWS, F_SIZE), f32)),
        deg_parts, x_pad, W1)

    agg1 = _sc_agg(h1, src3d, dst3d, zeros_f)
    x1, h2 = _tc_call(
        _tc_layer,
        (jax.ShapeDtypeStruct((NROWS, F_SIZE), f32),
         jax.ShapeDtypeStruct((NROWS, F_SIZE), f32)),
        agg1, h1, dinv, b1.reshape(1, F_SIZE), W2)

    agg2 = _sc_agg(h2, src3d, dst3d, zeros_f)
    x2, h3 = _tc_call(
        _tc_layer,
        (jax.ShapeDtypeStruct((NROWS, F_SIZE), f32),
         jax.ShapeDtypeStruct((NROWS, F_SIZE), f32)),
        agg2, h2, dinv, b2.reshape(1, F_SIZE), W3)

    agg3 = _sc_agg(h3, src3d, dst3d, zeros_f)
    out = _tc_call(
        _tc_final,
        jax.ShapeDtypeStruct((NUM_GRAPHS, OUT_SIZE), f32),
        agg3, h3, dinv, b3.reshape(1, F_SIZE), x1, x2,
        batch.astype(jnp.int32).reshape(1, N_NODES), Wf,
        bf.reshape(1, OUT_SIZE))
    return out


# speedup vs baseline: 1.3650x; 1.0074x over previous
import functools

import jax
import jax.numpy as jnp
from jax import lax
from jax.experimental import pallas as pl
from jax.experimental.pallas import tpu as pltpu
from jax.experimental.pallas import tpu_sc as plsc

N_NODES = 10000
N_EDGES = 320000
D_IN = 128
F_SIZE = 64
OUT_SIZE = 16
NUM_GRAPHS = 64

NC = 2
NS = 16
NW = NC * NS
CHUNK = 128
CPW = 80
EPO = 256
NBUF = 2
TOT_CH = 1280
NCH0 = 76
NCH1 = 4
NCHMAX = max(NCH0, NCH1)
E_PAD = NW * CPW * CHUNK
NROWS = 10112
ROWS_PER_TILE = NROWS // NS
DEG_W = 16

_MESH = plsc.VectorSubcoreMesh(core_axis_name="c", subcore_axis_name="s")


@functools.partial(
    pl.kernel,
    out_type=jax.ShapeDtypeStruct((NC, NROWS, DEG_W), jnp.float32),
    mesh=_MESH,
    scratch_types=[
        pltpu.VMEM((CPW, CHUNK), jnp.int32),
        pltpu.VMEM((CHUNK, DEG_W), jnp.float32),
        pltpu.VMEM_SHARED((NROWS, DEG_W), jnp.float32),
    ],
    compiler_params=pltpu.CompilerParams(use_tc_tiling_on_sc=False),
)
def _sc_degree(dst_hbm, ones_hbm, zeros_hbm, out_hbm, dst_v, ones_v, acc_sh):
    cid = lax.axis_index("c")
    sid = lax.axis_index("s")
    wid = cid * NS + sid
    pltpu.sync_copy(
        zeros_hbm.at[pl.ds(pl.multiple_of(sid * ROWS_PER_TILE, 8), ROWS_PER_TILE)],
        acc_sh.at[pl.ds(pl.multiple_of(sid * ROWS_PER_TILE, 8), ROWS_PER_TILE)],
    )
    pltpu.sync_copy(ones_hbm, ones_v)
    pltpu.sync_copy(dst_hbm.at[wid], dst_v)
    plsc.subcore_barrier()

    def body(j, carry):
        pltpu.sync_copy(ones_v, acc_sh.at[dst_v.at[j]], add=True)
        return carry

    lax.fori_loop(0, CPW, body, 0)
    plsc.subcore_barrier()
    pltpu.sync_copy(
        acc_sh.at[pl.ds(pl.multiple_of(sid * ROWS_PER_TILE, 8), ROWS_PER_TILE)],
        out_hbm.at[cid, pl.ds(pl.multiple_of(sid * ROWS_PER_TILE, 8), ROWS_PER_TILE)],
    )


@functools.partial(
    pl.kernel,
    out_type=jax.ShapeDtypeStruct((NC, NROWS, F_SIZE), jnp.float32),
    mesh=_MESH,
    scratch_types=[
        pltpu.VMEM((NCHMAX, EPO), jnp.int32),
        pltpu.VMEM((NCHMAX, EPO), jnp.int32),
        [pltpu.VMEM((EPO, F_SIZE), jnp.float32) for _ in range(NBUF)],
        pltpu.VMEM_SHARED((NROWS, F_SIZE), jnp.float32),
        [pltpu.SemaphoreType.DMA for _ in range(NBUF)],
        [pltpu.SemaphoreType.DMA for _ in range(NBUF)],
    ],
    compiler_params=pltpu.CompilerParams(use_tc_tiling_on_sc=False),
)
def _sc_agg(h_hbm, src_hbm, dst_hbm, zeros_hbm, out_hbm,
            src_v, dst_v, rows, acc_sh, gsem, ssem):
    cid = lax.axis_index("c")
    sid = lax.axis_index("s")
    pltpu.sync_copy(
        zeros_hbm.at[pl.ds(pl.multiple_of(sid * ROWS_PER_TILE, 8), ROWS_PER_TILE)],
        acc_sh.at[pl.ds(pl.multiple_of(sid * ROWS_PER_TILE, 8), ROWS_PER_TILE)],
    )

    def run(base, nch):
        pltpu.sync_copy(src_hbm.at[pl.ds(base, nch)], src_v.at[pl.ds(0, nch)])
        pltpu.sync_copy(dst_hbm.at[pl.ds(base, nch)], dst_v.at[pl.ds(0, nch)])
        for b in range(NBUF):
            pltpu.async_copy(h_hbm.at[src_v.at[b]], rows[b], gsem[b])

        def body(i, carry):
            for b in range(NBUF):
                j = NBUF * i + b
                pltpu.make_async_copy(
                    h_hbm.at[src_v.at[j]], rows[b], gsem[b]).wait()
                pltpu.async_copy(rows[b], acc_sh.at[dst_v.at[j]], ssem[b],
                                 add=True)
            for b in range(NBUF):
                j = NBUF * i + b

                @pl.when(j + NBUF < nch)
                def _(b=b, j=j):
                    pltpu.make_async_copy(
                        rows[b], acc_sh.at[dst_v.at[j]], ssem[b]).wait()
                    pltpu.async_copy(h_hbm.at[src_v.at[j + NBUF]], rows[b],
                                     gsem[b])

            return carry

        lax.fori_loop(0, nch // NBUF, body, 0)
        for b in range(NBUF):
            j = nch - NBUF + b
            pltpu.make_async_copy(rows[b], acc_sh.at[dst_v.at[j]],
                                  ssem[b]).wait()

    if NCH0:
        @pl.when(cid == 0)
        def _():
            run(pl.multiple_of(sid * NCH0, 8), NCH0)

    if NCH1:
        @pl.when(cid == 1)
        def _():
            run(pl.multiple_of(NS * NCH0 + sid * NCH1, 8), NCH1)

    plsc.subcore_barrier()
    pltpu.sync_copy(
        acc_sh.at[pl.ds(pl.multiple_of(sid * ROWS_PER_TILE, 8), ROWS_PER_TILE)],
        out_hbm.at[cid, pl.ds(pl.multiple_of(sid * ROWS_PER_TILE, 8), ROWS_PER_TILE)],
    )


def _tc_prologue(deg_ref, x_ref, w1_ref, dinv_ref, h1_ref):
    deg = deg_ref[0, :, 0:1] + deg_ref[1, :, 0:1] + 1.0
    rows = lax.broadcasted_iota(jnp.int32, (NROWS, 1), 0)
    dinv = jnp.where(rows < N_NODES, lax.rsqrt(deg), 0.0)
    dinv_ref[...] = dinv
    h1_ref[...] = jnp.dot(x_ref[...], w1_ref[...],
                          preferred_element_type=jnp.float32) * dinv


def _tc_layer(agg_ref, h_ref, dinv_ref, b_ref, wn_ref, x_ref, hn_ref):
    dinv = dinv_ref[...]
    out = (agg_ref[0] + agg_ref[1] + h_ref[...]) * dinv + b_ref[...]
    x = jnp.maximum(out, 0.0)
    x_ref[...] = x
    hn_ref[...] = jnp.dot(x, wn_ref[...],
                          preferred_element_type=jnp.float32) * dinv


def _tc_final(agg_ref, h3_ref, dinv_ref, b3_ref, x1_ref, x2_ref,
              batch_ref, wf_ref, bf_ref, out_ref):
    dinv = dinv_ref[...]
    x3 = jnp.maximum(
        (agg_ref[0] + agg_ref[1] + h3_ref[...]) * dinv + b3_ref[...], 0.0)
    onehot = (batch_ref[...] ==
              lax.broadcasted_iota(jnp.int32, (NUM_GRAPHS, N_NODES), 0))
    seg = jnp.where(onehot, 1.0, 0.0)
    counts = jnp.sum(seg, axis=1, keepdims=True)
    hcat = jnp.concatenate(
        [x1_ref[:N_NODES], x2_ref[:N_NODES], x3[:N_NODES]], axis=1)
    pooled = jnp.dot(seg, hcat, preferred_element_type=jnp.float32)
    pooled = pooled / jnp.maximum(counts, 1.0)
    logits = jnp.dot(pooled, wf_ref[...],
                     preferred_element_type=jnp.float32) + bf_ref[...]
    m = jnp.max(logits, axis=1, keepdims=True)
    e = jnp.exp(logits - m)
    out_ref[...] = e / jnp.sum(e, axis=1, keepdims=True)


def _tc_call(body, out_shape, *args):
    return pl.pallas_call(body, out_shape=out_shape)(*args)


def kernel(x, edge_index, batch, W1, b1, W2, b2, W3, b3, Wf, bf):
    src = edge_index[0].astype(jnp.int32)
    dst = edge_index[1].astype(jnp.int32)
    pad = jnp.full((E_PAD - N_EDGES,), N_NODES, jnp.int32)
    src_pad = jnp.concatenate([src, pad])
    dst_pad = jnp.concatenate([dst, pad])
    src3d = src_pad.reshape(TOT_CH, EPO)
    dst3d = dst_pad.reshape(TOT_CH, EPO)
    dst2d = dst_pad.reshape(NW, CPW, CHUNK)

    x_pad = jnp.zeros((NROWS, D_IN), jnp.float32).at[:N_NODES].set(x)
    zeros_deg = jnp.zeros((NROWS, DEG_W), jnp.float32)
    zeros_f = jnp.zeros((NROWS, F_SIZE), jnp.float32)
    ones_rows = jnp.ones((CHUNK, DEG_W), jnp.float32)

    deg_parts = _sc_degree(dst2d, ones_rows, zeros_deg)

    f32 = jnp.float32
    dinv, h1 = _tc_call(
        _tc_prologue,
        (jax.ShapeDtypeStruct((NROWS, 1), f32),
         jax.ShapeDtypeStruct((NROWS, F_SIZE), f32)),
        deg_parts, x_pad, W1)

    agg1 = _sc_agg(h1, src3d, dst3d, zeros_f)
    x1, h2 = _tc_call(
        _tc_layer,
        (jax.ShapeDtypeStruct((NROWS, F_SIZE), f32),
         jax.ShapeDtypeStruct((NROWS, F_SIZE), f32)),
        agg1, h1, dinv, b1.reshape(1, F_SIZE), W2)

    agg2 = _sc_agg(h2, src3d, dst3d, zeros_f)
    x2, h3 = _tc_call(
        _tc_layer,
        (jax.ShapeDtypeStruct((NROWS, F_SIZE), f32),
         jax.ShapeDtypeStruct((NROWS, F_SIZE), f32)),
        agg2, h2, dinv, b2.reshape(1, F_SIZE), W3)

    agg3 = _sc_agg(h3, src3d, dst3d, zeros_f)
    out = _tc_call(
        _tc_final,
        jax.ShapeDtypeStruct((NUM_GRAPHS, OUT_SIZE), f32),
        agg3, h3, dinv, b3.reshape(1, F_SIZE), x1, x2,
        batch.astype(jnp.int32).reshape(1, N_NODES), Wf,
        bf.reshape(1, OUT_SIZE))
    return out


# speedup vs baseline: 2.0323x; 1.4888x over previous
import functools

import jax
import jax.numpy as jnp
from jax import lax
from jax.experimental import pallas as pl
from jax.experimental.pallas import tpu as pltpu
from jax.experimental.pallas import tpu_sc as plsc

N_NODES = 10000
N_EDGES = 320000
D_IN = 128
F_SIZE = 64
OUT_SIZE = 16
NUM_GRAPHS = 64

NC = 2
NS = 16
NW = NC * NS
CHUNK = 128
CPW = 80
EPO = 256
NBUF = 2
TOT_CH = 1280
NCH0 = 76
NCH1 = 4
NCHMAX = max(NCH0, NCH1)
E_PAD = NW * CPW * CHUNK
NROWS = 10112
ROWS_PER_TILE = NROWS // NS
DEG_W = 16

_MESH = plsc.VectorSubcoreMesh(core_axis_name="c", subcore_axis_name="s")


@functools.partial(
    pl.kernel,
    out_type=jax.ShapeDtypeStruct((NC, NROWS, DEG_W), jnp.float32),
    mesh=_MESH,
    scratch_types=[
        pltpu.VMEM((CPW, CHUNK), jnp.int32),
        pltpu.VMEM((CHUNK, DEG_W), jnp.float32),
        pltpu.VMEM_SHARED((NROWS, DEG_W), jnp.float32),
    ],
    compiler_params=pltpu.CompilerParams(use_tc_tiling_on_sc=False),
)
def _sc_degree(dst_hbm, ones_hbm, zeros_hbm, out_hbm, dst_v, ones_v, acc_sh):
    cid = lax.axis_index("c")
    sid = lax.axis_index("s")
    wid = cid * NS + sid
    pltpu.sync_copy(
        zeros_hbm.at[pl.ds(pl.multiple_of(sid * ROWS_PER_TILE, 8), ROWS_PER_TILE)],
        acc_sh.at[pl.ds(pl.multiple_of(sid * ROWS_PER_TILE, 8), ROWS_PER_TILE)],
    )
    pltpu.sync_copy(ones_hbm, ones_v)
    pltpu.sync_copy(dst_hbm.at[wid], dst_v)
    plsc.subcore_barrier()

    def body(j, carry):
        pltpu.sync_copy(ones_v, acc_sh.at[dst_v.at[j]], add=True)
        return carry

    lax.fori_loop(0, CPW, body, 0)
    plsc.subcore_barrier()
    pltpu.sync_copy(
        acc_sh.at[pl.ds(pl.multiple_of(sid * ROWS_PER_TILE, 8), ROWS_PER_TILE)],
        out_hbm.at[cid, pl.ds(pl.multiple_of(sid * ROWS_PER_TILE, 8), ROWS_PER_TILE)],
    )


@functools.partial(
    pl.kernel,
    out_type=jax.ShapeDtypeStruct((NC, NROWS, F_SIZE), jnp.bfloat16),
    mesh=_MESH,
    scratch_types=[
        pltpu.VMEM((NCHMAX, EPO), jnp.int32),
        pltpu.VMEM((NCHMAX, EPO), jnp.int32),
        [pltpu.VMEM((EPO, F_SIZE), jnp.bfloat16) for _ in range(NBUF)],
        pltpu.VMEM_SHARED((NROWS, F_SIZE), jnp.bfloat16),
        [pltpu.SemaphoreType.DMA for _ in range(NBUF)],
        [pltpu.SemaphoreType.DMA for _ in range(NBUF)],
    ],
    compiler_params=pltpu.CompilerParams(use_tc_tiling_on_sc=False),
)
def _sc_agg(h_hbm, src_hbm, dst_hbm, zeros_hbm, out_hbm,
            src_v, dst_v, rows, acc_sh, gsem, ssem):
    cid = lax.axis_index("c")
    sid = lax.axis_index("s")
    pltpu.sync_copy(
        zeros_hbm.at[pl.ds(pl.multiple_of(sid * ROWS_PER_TILE, 8), ROWS_PER_TILE)],
        acc_sh.at[pl.ds(pl.multiple_of(sid * ROWS_PER_TILE, 8), ROWS_PER_TILE)],
    )

    def run(base, nch):
        pltpu.sync_copy(src_hbm.at[pl.ds(base, nch)], src_v.at[pl.ds(0, nch)])
        pltpu.sync_copy(dst_hbm.at[pl.ds(base, nch)], dst_v.at[pl.ds(0, nch)])
        for b in range(NBUF):
            pltpu.async_copy(h_hbm.at[src_v.at[b]], rows[b], gsem[b])

        def body(i, carry):
            for b in range(NBUF):
                j = NBUF * i + b
                pltpu.make_async_copy(
                    h_hbm.at[src_v.at[j]], rows[b], gsem[b]).wait()
                pltpu.async_copy(rows[b], acc_sh.at[dst_v.at[j]], ssem[b],
                                 add=True)
            for b in range(NBUF):
                j = NBUF * i + b

                @pl.when(j + NBUF < nch)
                def _(b=b, j=j):
                    pltpu.make_async_copy(
                        rows[b], acc_sh.at[dst_v.at[j]], ssem[b]).wait()
                    pltpu.async_copy(h_hbm.at[src_v.at[j + NBUF]], rows[b],
                                     gsem[b])

            return carry

        lax.fori_loop(0, nch // NBUF, body, 0)
        for b in range(NBUF):
            j = nch - NBUF + b
            pltpu.make_async_copy(rows[b], acc_sh.at[dst_v.at[j]],
                                  ssem[b]).wait()

    if NCH0:
        @pl.when(cid == 0)
        def _():
            run(pl.multiple_of(sid * NCH0, 8), NCH0)

    if NCH1:
        @pl.when(cid == 1)
        def _():
            run(pl.multiple_of(NS * NCH0 + sid * NCH1, 8), NCH1)

    plsc.subcore_barrier()
    pltpu.sync_copy(
        acc_sh.at[pl.ds(pl.multiple_of(sid * ROWS_PER_TILE, 8), ROWS_PER_TILE)],
        out_hbm.at[cid, pl.ds(pl.multiple_of(sid * ROWS_PER_TILE, 8), ROWS_PER_TILE)],
    )


def _tc_prologue(deg_ref, x_ref, w1_ref, dinv_ref, h1_ref, h1b_ref):
    deg = deg_ref[0, :, 0:1] + deg_ref[1, :, 0:1] + 1.0
    rows = lax.broadcasted_iota(jnp.int32, (NROWS, 1), 0)
    dinv = jnp.where(rows < N_NODES, lax.rsqrt(deg), 0.0)
    dinv_ref[...] = dinv
    h1 = jnp.dot(x_ref[...], w1_ref[...],
                 preferred_element_type=jnp.float32) * dinv
    h1_ref[...] = h1
    h1b_ref[...] = h1.astype(jnp.bfloat16)


def _tc_layer(agg_ref, h_ref, dinv_ref, b_ref, wn_ref, x_ref, hn_ref, hnb_ref):
    dinv = dinv_ref[...]
    agg = (agg_ref[0].astype(jnp.float32) + agg_ref[1].astype(jnp.float32))
    out = (agg + h_ref[...]) * dinv + b_ref[...]
    x = jnp.maximum(out, 0.0)
    x_ref[...] = x
    hn = jnp.dot(x, wn_ref[...], preferred_element_type=jnp.float32) * dinv
    hn_ref[...] = hn
    hnb_ref[...] = hn.astype(jnp.bfloat16)


def _tc_final(agg_ref, h3_ref, dinv_ref, b3_ref, x1_ref, x2_ref,
              batch_ref, wf_ref, bf_ref, out_ref):
    dinv = dinv_ref[...]
    agg = (agg_ref[0].astype(jnp.float32) + agg_ref[1].astype(jnp.float32))
    x3 = jnp.maximum(agg * dinv + h3_ref[...] * dinv + b3_ref[...], 0.0)
    onehot = (batch_ref[...] ==
              lax.broadcasted_iota(jnp.int32, (NUM_GRAPHS, N_NODES), 0))
    seg = jnp.where(onehot, 1.0, 0.0)
    counts = jnp.sum(seg, axis=1, keepdims=True)
    hcat = jnp.concatenate(
        [x1_ref[:N_NODES], x2_ref[:N_NODES], x3[:N_NODES]], axis=1)
    pooled = jnp.dot(seg, hcat, preferred_element_type=jnp.float32)
    pooled = pooled / jnp.maximum(counts, 1.0)
    logits = jnp.dot(pooled, wf_ref[...],
                     preferred_element_type=jnp.float32) + bf_ref[...]
    m = jnp.max(logits, axis=1, keepdims=True)
    e = jnp.exp(logits - m)
    out_ref[...] = e / jnp.sum(e, axis=1, keepdims=True)


def _tc_call(body, out_shape, *args):
    return pl.pallas_call(body, out_shape=out_shape)(*args)


def kernel(x, edge_index, batch, W1, b1, W2, b2, W3, b3, Wf, bf):
    src = edge_index[0].astype(jnp.int32)
    dst = edge_index[1].astype(jnp.int32)
    pad = jnp.full((E_PAD - N_EDGES,), N_NODES, jnp.int32)
    src_pad = jnp.concatenate([src, pad])
    dst_pad = jnp.concatenate([dst, pad])
    src3d = src_pad.reshape(TOT_CH, EPO)
    dst3d = dst_pad.reshape(TOT_CH, EPO)
    dst2d = dst_pad.reshape(NW, CPW, CHUNK)

    x_pad = jnp.zeros((NROWS, D_IN), jnp.float32).at[:N_NODES].set(x)
    zeros_deg = jnp.zeros((NROWS, DEG_W), jnp.float32)
    zeros_f = jnp.zeros((NROWS, F_SIZE), jnp.bfloat16)
    ones_rows = jnp.ones((CHUNK, DEG_W), jnp.float32)

    deg_parts = _sc_degree(dst2d, ones_rows, zeros_deg)

    f32 = jnp.float32
    dinv, h1, h1b = _tc_call(
        _tc_prologue,
        (jax.ShapeDtypeStruct((NROWS, 1), f32),
         jax.ShapeDtypeStruct((NROWS, F_SIZE), f32),
         jax.ShapeDtypeStruct((NROWS, F_SIZE), jnp.bfloat16)),
        deg_parts, x_pad, W1)

    agg1 = _sc_agg(h1b, src3d, dst3d, zeros_f)
    x1, h2, h2b = _tc_call(
        _tc_layer,
        (jax.ShapeDtypeStruct((NROWS, F_SIZE), f32),
         jax.ShapeDtypeStruct((NROWS, F_SIZE), f32),
         jax.ShapeDtypeStruct((NROWS, F_SIZE), jnp.bfloat16)),
        agg1, h1, dinv, b1.reshape(1, F_SIZE), W2)

    agg2 = _sc_agg(h2b, src3d, dst3d, zeros_f)
    x2, h3, h3b = _tc_call(
        _tc_layer,
        (jax.ShapeDtypeStruct((NROWS, F_SIZE), f32),
         jax.ShapeDtypeStruct((NROWS, F_SIZE), f32),
         jax.ShapeDtypeStruct((NROWS, F_SIZE), jnp.bfloat16)),
        agg2, h2, dinv, b2.reshape(1, F_SIZE), W3)

    agg3 = _sc_agg(h3b, src3d, dst3d, zeros_f)
    out = _tc_call(
        _tc_final,
        jax.ShapeDtypeStruct((NUM_GRAPHS, OUT_SIZE), f32),
        agg3, h3, dinv, b3.reshape(1, F_SIZE), x1, x2,
        batch.astype(jnp.int32).reshape(1, N_NODES), Wf,
        bf.reshape(1, OUT_SIZE))
    return out
